# baseline (speedup 1.0000x reference)
import functools

import jax
import jax.numpy as jnp
from jax import lax
from jax.experimental import pallas as pl
from jax.experimental.pallas import tpu as pltpu
from jax.experimental.pallas import tpu_sc as plsc

NC = 2
NS = 16
NW = NC * NS
LANES = 16
NBUF = 2


def _make_sc_add(B, S, D):
    SP = S // NW
    CHUNKS = D // LANES
    mesh = plsc.VectorSubcoreMesh(core_axis_name="c", subcore_axis_name="s")

    @functools.partial(
        pl.kernel,
        mesh=mesh,
        out_type=jax.ShapeDtypeStruct((B, S, D), jnp.float32),
        scratch_types=[
            pltpu.VMEM((SP, D), jnp.float32),
            pltpu.VMEM((NBUF, SP, D), jnp.float32),
            pltpu.VMEM((NBUF, SP, D), jnp.float32),
            pltpu.SemaphoreType.DMA,
            pltpu.SemaphoreType.DMA,
            pltpu.SemaphoreType.DMA,
            pltpu.SemaphoreType.DMA,
        ],
    )
    def sc_add(patch_hbm, pos_hbm, out_hbm, pos_v, ibuf, obuf, in0, in1, out0, out1):
        in_sems = (in0, in1)
        out_sems = (out0, out1)
        wid = lax.axis_index("s") * NC + lax.axis_index("c")
        s0 = wid * SP

        pltpu.sync_copy(pos_hbm.at[pl.ds(s0, SP)], pos_v)

        def start_in(j, b):
            pltpu.async_copy(patch_hbm.at[b, pl.ds(s0, SP)], ibuf.at[j], in_sems[j])

        def wait_in(j, b):
            pltpu.make_async_copy(
                patch_hbm.at[b, pl.ds(s0, SP)], ibuf.at[j], in_sems[j]
            ).wait()

        def start_out(j, b):
            pltpu.async_copy(obuf.at[j], out_hbm.at[b, pl.ds(s0, SP)], out_sems[j])

        def wait_out(j, b):
            pltpu.make_async_copy(
                obuf.at[j], out_hbm.at[b, pl.ds(s0, SP)], out_sems[j]
            ).wait()

        def compute(j):
            ib = ibuf.at[j]
            ob = obuf.at[j]

            @plsc.parallel_loop(0, SP)
            def _(r):
                for c in range(CHUNKS):
                    sl = pl.ds(c * LANES, LANES)
                    ob[r, sl] = ib[r, sl] + pos_v[r, sl]

        R = B // NBUF

        for j in range(NBUF):
            start_in(j, j)

        for j in range(NBUF):
            wait_in(j, j)
            compute(j)
            start_out(j, j)
            start_in(j, j + NBUF)

        def round_body(rb, _):
            for j in range(NBUF):
                b = rb * NBUF + j
                wait_in(j, b)
                wait_out(j, b - NBUF)
                compute(j)
                start_out(j, b)
                start_in(j, b + NBUF)
            return 0

        lax.fori_loop(1, R - 1, round_body, 0)

        for j in range(NBUF):
            b = (R - 1) * NBUF + j
            wait_in(j, b)
            wait_out(j, b - NBUF)
            compute(j)
            start_out(j, b)

        for j in range(NBUF):
            wait_out(j, (R - 1) * NBUF + j)

    return sc_add


def kernel(patch, position_embedding):
    B, S, D = patch.shape
    pos = position_embedding[:S]
    return _make_sc_add(B, S, D)(patch, pos)
